# baseline (speedup 1.0000x reference)
import functools

import jax
import jax.numpy as jnp
from jax import lax
from jax.experimental import pallas as pl
from jax.experimental.pallas import tpu as pltpu
from jax.experimental.pallas import tpu_sc as plsc

CHUNK = 2048
MAXP = 7
LANES = 16


def _sc_routing_body(num_chunks, zrow, batch, n_hbm, m_hbm, out_hbm, n_v, m_v, out_v):
    @pl.when((lax.axis_index("c") == 0) & (lax.axis_index("s") == 0))
    def _():
        pltpu.sync_copy(n_hbm, n_v)
        pltpu.sync_copy(m_hbm, m_v)
        n = n_v[...]
        lane = lax.iota(jnp.int32, LANES)
        start = jnp.zeros((LANES,), jnp.int32)
        for k in range(batch):
            bk = plsc.load_gather(n_v, [jnp.full((LANES,), batch + k, jnp.int32)])
            start = start + jnp.where(lane > k, bk + 1, 0)
        start = jnp.clip(start, 0, num_chunks)
        npk = jnp.minimum(n, m_v[...])
        for j in range(MAXP):
            valid = (npk > j) & (lane < batch)
            out_v[j, :] = jnp.where(valid, start + j, zrow)
        pltpu.sync_copy(out_v, out_hbm)


def _tc_body(num_chunks, zrow, batch, cpb, src_ref, scale_ref, x_ref, out_ref, means):
    c = pl.program_id(0)

    @pl.when(c == 0)
    def _():
        means[pl.ds(num_chunks, zrow + 1 - num_chunks), :] = jnp.zeros(
            (zrow + 1 - num_chunks, means.shape[1]), jnp.float32
        )

    blk = x_ref[...].reshape(cpb, CHUNK, means.shape[1])
    sums = jnp.sum(blk, axis=1) * scale_ref[0]
    for k in range(cpb):
        means[pl.ds(c * cpb + k, 1), :] = sums[k : k + 1]

    @pl.when(c == num_chunks // cpb - 1)
    def _():
        for i in range(batch):
            for j in range(MAXP):
                v = src_ref[j * LANES + i]
                out_ref[pl.ds(i, 1), pl.ds(j, 1), :] = means[pl.ds(v, 1), :].reshape(
                    1, 1, means.shape[1]
                )


def kernel(x, chunk_size, n_peaks, max_n_peaks):
    batch, length, embed = x.shape
    xf = x.reshape(-1, embed)
    num_chunks = xf.shape[0] // CHUNK
    zrow = num_chunks + MAXP

    n32 = n_peaks.astype(jnp.int32)
    n_pad = (
        jnp.zeros((LANES,), jnp.int32)
        .at[:batch]
        .set(n32)
        .at[batch : 2 * batch]
        .set(n32)
    )
    maxv = jnp.full((LANES,), max_n_peaks, dtype=jnp.int32)

    mesh = plsc.VectorSubcoreMesh(core_axis_name="c", subcore_axis_name="s")
    src = pl.kernel(
        functools.partial(_sc_routing_body, num_chunks, zrow, batch),
        out_type=jax.ShapeDtypeStruct((MAXP, LANES), jnp.int32),
        mesh=mesh,
        compiler_params=pltpu.CompilerParams(needs_layout_passes=False),
        scratch_types=[
            pltpu.VMEM((LANES,), jnp.int32),
            pltpu.VMEM((LANES,), jnp.int32),
            pltpu.VMEM((MAXP, LANES), jnp.int32),
        ],
    )(n_pad, maxv)

    scale = (1.0 / jnp.asarray(chunk_size, jnp.float32)).reshape(1)

    cpb = 3
    out = pl.pallas_call(
        functools.partial(_tc_body, num_chunks, zrow, batch, cpb),
        grid=(num_chunks // cpb,),
        in_specs=[
            pl.BlockSpec(memory_space=pltpu.SMEM),
            pl.BlockSpec(memory_space=pltpu.SMEM),
            pl.BlockSpec((cpb * CHUNK, embed), lambda c: (c, 0)),
        ],
        out_specs=pl.BlockSpec((batch, MAXP, embed), lambda c: (0, 0, 0)),
        out_shape=jax.ShapeDtypeStruct((batch, MAXP, embed), jnp.float32),
        scratch_shapes=[pltpu.VMEM((zrow + 1, embed), jnp.float32)],
    )(src.reshape(-1), scale, xf)
    return out


# speedup vs baseline: 1.0364x; 1.0364x over previous
import functools

import jax
import jax.numpy as jnp
from jax import lax
from jax.experimental import pallas as pl
from jax.experimental.pallas import tpu as pltpu
from jax.experimental.pallas import tpu_sc as plsc

CHUNK = 2048
MAXP = 7
LANES = 16


def _sc_routing_body(num_chunks, zrow, batch, n_hbm, m_hbm, out_hbm, n_v, m_v, out_v):
    @pl.when((lax.axis_index("c") == 0) & (lax.axis_index("s") == 0))
    def _():
        pltpu.sync_copy(n_hbm, n_v)
        pltpu.sync_copy(m_hbm, m_v)
        n = n_v[...]
        lane = lax.iota(jnp.int32, LANES)
        start = jnp.zeros((LANES,), jnp.int32)
        for k in range(batch):
            bk = plsc.load_gather(n_v, [jnp.full((LANES,), batch + k, jnp.int32)])
            start = start + jnp.where(lane > k, bk + 1, 0)
        start = jnp.clip(start, 0, num_chunks)
        npk = jnp.minimum(n, m_v[...])
        for j in range(MAXP):
            valid = (npk > j) & (lane < batch)
            out_v[j, :] = jnp.where(valid, start + j, zrow)
        pltpu.sync_copy(out_v, out_hbm)


def _tc_body(num_chunks, zrow, batch, cpb, src_ref, scale_ref, x_ref, out_ref, means):
    c = pl.program_id(0)

    @pl.when(c == 0)
    def _():
        means[pl.ds(num_chunks, zrow + 1 - num_chunks), :] = jnp.zeros(
            (zrow + 1 - num_chunks, means.shape[1]), jnp.float32
        )

    blk = x_ref[...].reshape(cpb, CHUNK, means.shape[1])
    sums = jnp.sum(blk, axis=1) * scale_ref[0]
    for k in range(cpb):
        means[pl.ds(c * cpb + k, 1), :] = sums[k : k + 1]

    @pl.when(c == num_chunks // cpb - 1)
    def _():
        for i in range(batch):
            for j in range(MAXP):
                v = src_ref[j * LANES + i]
                out_ref[pl.ds(i, 1), pl.ds(j, 1), :] = means[pl.ds(v, 1), :].reshape(
                    1, 1, means.shape[1]
                )


def kernel(x, chunk_size, n_peaks, max_n_peaks):
    batch, length, embed = x.shape
    xf = x.reshape(-1, embed)
    num_chunks = xf.shape[0] // CHUNK
    zrow = num_chunks + MAXP

    n32 = n_peaks.astype(jnp.int32)
    n_pad = (
        jnp.zeros((LANES,), jnp.int32)
        .at[:batch]
        .set(n32)
        .at[batch : 2 * batch]
        .set(n32)
    )
    maxv = jnp.full((LANES,), max_n_peaks, dtype=jnp.int32)

    mesh = plsc.VectorSubcoreMesh(core_axis_name="c", subcore_axis_name="s")
    src = pl.kernel(
        functools.partial(_sc_routing_body, num_chunks, zrow, batch),
        out_type=jax.ShapeDtypeStruct((MAXP, LANES), jnp.int32),
        mesh=mesh,
        compiler_params=pltpu.CompilerParams(needs_layout_passes=False),
        scratch_types=[
            pltpu.VMEM((LANES,), jnp.int32),
            pltpu.VMEM((LANES,), jnp.int32),
            pltpu.VMEM((MAXP, LANES), jnp.int32),
        ],
    )(n_pad, maxv)

    scale = (1.0 / jnp.asarray(chunk_size, jnp.float32)).reshape(1)

    cpb = 2
    out = pl.pallas_call(
        functools.partial(_tc_body, num_chunks, zrow, batch, cpb),
        grid=(num_chunks // cpb,),
        in_specs=[
            pl.BlockSpec(memory_space=pltpu.SMEM),
            pl.BlockSpec(memory_space=pltpu.SMEM),
            pl.BlockSpec((cpb * CHUNK, embed), lambda c: (c, 0)),
        ],
        out_specs=pl.BlockSpec((batch, MAXP, embed), lambda c: (0, 0, 0)),
        out_shape=jax.ShapeDtypeStruct((batch, MAXP, embed), jnp.float32),
        scratch_shapes=[pltpu.VMEM((zrow + 1, embed), jnp.float32)],
    )(src.reshape(-1), scale, xf)
    return out
